# baseline (speedup 1.0000x reference)
import jax
import jax.numpy as jnp
from jax import lax
from jax.experimental import pallas as pl
from jax.experimental.pallas import tpu as pltpu

M, N, DIM = 32, 32, 256
K = M * N
B = 256
T2_INV = 1.0 / (100.0 * 100.0)


def _som_body(x_ref, w_ref, wout_ref, loss_ref, sem):
    x = x_ref[...]
    w = w_ref[...]
    wcopy = pltpu.make_async_copy(w_ref, wout_ref, sem)
    wcopy.start()

    xwn = lax.dot_general(
        -2.0 * x, w, (((1,), (1,)), ((), ())),
        preferred_element_type=jnp.float32,
    )
    w2 = lax.dot_general(
        jnp.ones((1, DIM), jnp.float32), w * w,
        (((1,), (1,)), ((), ())),
        preferred_element_type=jnp.float32,
    )
    x2 = jnp.sum(x * x, axis=1, keepdims=True)
    score = w2 + xwn
    dist = score + x2

    bmu = jnp.argmin(score, axis=1).reshape(B, 1)

    bi = (bmu >> 5).astype(jnp.float32)
    bj = (bmu & 31).astype(jnp.float32)
    krow = lax.broadcasted_iota(jnp.int32, (1, K), 1)
    ki = (krow >> 5).astype(jnp.float32)
    kj = (krow & 31).astype(jnp.float32)

    man = jnp.abs(ki - bi) + jnp.abs(kj - bj)
    infl = jnp.exp(-(man * man) * T2_INV)
    rowsum = jnp.sum(dist * infl, axis=1, keepdims=True)
    loss_ref[...] = jnp.sum(rowsum, axis=0, keepdims=True) * (1.0 / N)
    wcopy.wait()


def kernel(inputs, som_weights, locations):
    w_out, loss = pl.pallas_call(
        _som_body,
        out_specs=(
            pl.BlockSpec(memory_space=pl.ANY),
            pl.BlockSpec(memory_space=pltpu.MemorySpace.VMEM),
        ),
        out_shape=(
            jax.ShapeDtypeStruct((K, DIM), jnp.float32),
            jax.ShapeDtypeStruct((1, 1), jnp.float32),
        ),
        scratch_shapes=[pltpu.SemaphoreType.DMA],
    )(inputs, som_weights)
    return w_out, loss.reshape(())
